# baseline (speedup 1.0000x reference)
import functools

import jax
import jax.numpy as jnp
from jax import lax
from jax.experimental import pallas as pl
from jax.experimental.pallas import tpu as pltpu
from jax.experimental.pallas import tpu_sc as plsc

N = 10000
D = 128
NP = 10240
E = 320000
NS = 16
BLK = 128
NBT = 158
E_PAD = NS * NBT * BLK
BM = 512
GRID = NP // BM
K0_96, K0_48 = 97, 93
K1_96, K1_48 = NBT - K0_96, NBT - K0_48


def _k1a_body(x_ref, wa_ref, ba_ref, y_ref):
    y_ref[...] = jnp.dot(x_ref[...], wa_ref[...],
                         preferred_element_type=jnp.float32) + ba_ref[...]


def _k1a(x_p, W_aug, b_aug):
    return pl.pallas_call(
        _k1a_body,
        grid=(GRID,),
        in_specs=[
            pl.BlockSpec((BM, D), lambda i: (i, 0)),
            pl.BlockSpec((D, 96), lambda i: (0, 0)),
            pl.BlockSpec((1, 96), lambda i: (0, 0)),
        ],
        out_specs=pl.BlockSpec((BM, 96), lambda i: (i, 0)),
        out_shape=jax.ShapeDtypeStruct((NP, 96), jnp.float32),
    )(x_p, W_aug, b_aug)


def _k1b_body(x_ref, w0_ref, b0_ref, h0_ref):
    h0_ref[...] = jnp.maximum(
        jnp.dot(x_ref[...], w0_ref[...], preferred_element_type=jnp.float32)
        + b0_ref[...], 0.0)


def _k1b(x_p, W0p, b0p):
    return pl.pallas_call(
        _k1b_body,
        grid=(GRID,),
        in_specs=[
            pl.BlockSpec((BM, D), lambda i: (i, 0)),
            pl.BlockSpec((D, 48), lambda i: (0, 0)),
            pl.BlockSpec((1, 48), lambda i: (0, 0)),
        ],
        out_specs=pl.BlockSpec((BM, 48), lambda i: (i, 0)),
        out_shape=jax.ShapeDtypeStruct((NP, 48), jnp.float32),
    )(x_p, W0p, b0p)


def _k3a_body(zp_ref, z_ref):
    z_ref[...] = zp_ref[0] + zp_ref[1]


def _k3a(zp):
    return pl.pallas_call(
        _k3a_body,
        grid=(GRID,),
        in_specs=[pl.BlockSpec((2, BM, 96), lambda i: (0, i, 0))],
        out_specs=pl.BlockSpec((BM, 96), lambda i: (i, 0)),
        out_shape=jax.ShapeDtypeStruct((NP, 96), jnp.float32),
    )(zp)


def _k3b_body(z_ref, b1p_ref, h1r_ref, d2b_ref):
    z = z_ref[...]
    dinv = 1.0 / jnp.maximum(z[:, 95:96], 1.0)
    h1r_ref[...] = jnp.maximum(z * dinv + b1p_ref[...], 0.0)
    d2b_ref[...] = jnp.broadcast_to(dinv * dinv, (BM, 48))


def _k3b(z, b1p):
    return pl.pallas_call(
        _k3b_body,
        grid=(GRID,),
        in_specs=[
            pl.BlockSpec((BM, 96), lambda i: (i, 0)),
            pl.BlockSpec((1, 96), lambda i: (0, 0)),
        ],
        out_specs=[
            pl.BlockSpec((BM, 96), lambda i: (i, 0)),
            pl.BlockSpec((BM, 48), lambda i: (i, 0)),
        ],
        out_shape=[
            jax.ShapeDtypeStruct((NP, 96), jnp.float32),
            jax.ShapeDtypeStruct((NP, 48), jnp.float32),
        ],
    )(z, b1p)


def _k5a_body(h0r_ref, h1r_ref, wf0_ref, wf1_ref, bf_ref, s_ref):
    s_ref[...] = (
        jnp.dot(h0r_ref[...], wf0_ref[...], preferred_element_type=jnp.float32)
        + jnp.dot(h1r_ref[...], wf1_ref[...], preferred_element_type=jnp.float32)
        + bf_ref[...])


def _k5a(h0r, h1r, Wf0p, Wf1p, bfp):
    return pl.pallas_call(
        _k5a_body,
        grid=(GRID,),
        in_specs=[
            pl.BlockSpec((BM, 48), lambda i: (i, 0)),
            pl.BlockSpec((BM, 96), lambda i: (i, 0)),
            pl.BlockSpec((48, 8), lambda i: (0, 0)),
            pl.BlockSpec((96, 8), lambda i: (0, 0)),
            pl.BlockSpec((1, 8), lambda i: (0, 0)),
        ],
        out_specs=pl.BlockSpec((BM, 8), lambda i: (i, 0)),
        out_shape=jax.ShapeDtypeStruct((NP, 8), jnp.float32),
    )(h0r, h1r, Wf0p, Wf1p, bfp)


def _k5b_body(s_ref, tp_ref, d2b_ref, b2p_ref, wf2_ref, out_ref):
    t = tp_ref[0] + tp_ref[1]
    h2r = jnp.maximum(t * d2b_ref[...] + b2p_ref[...], 0.0)
    out_ref[...] = s_ref[...] + jnp.dot(
        h2r, wf2_ref[...], preferred_element_type=jnp.float32)


def _k5b(s01, tp, d2b, b2p, Wf2p):
    return pl.pallas_call(
        _k5b_body,
        grid=(GRID,),
        in_specs=[
            pl.BlockSpec((BM, 8), lambda i: (i, 0)),
            pl.BlockSpec((2, BM, 48), lambda i: (0, i, 0)),
            pl.BlockSpec((BM, 48), lambda i: (i, 0)),
            pl.BlockSpec((1, 48), lambda i: (0, 0)),
            pl.BlockSpec((48, 8), lambda i: (0, 0)),
        ],
        out_specs=pl.BlockSpec((BM, 8), lambda i: (i, 0)),
        out_shape=jax.ShapeDtypeStruct((NP, 8), jnp.float32),
    )(s01, tp, d2b, b2p, Wf2p)


def _make_spmm(width, k0, k1):
    rows_per = NP // NS
    kmax = max(k0, k1)
    mesh = plsc.VectorSubcoreMesh(core_axis_name="c", subcore_axis_name="s")

    @functools.partial(
        pl.kernel,
        mesh=mesh,
        compiler_params=pltpu.CompilerParams(use_tc_tiling_on_sc=False),
        out_type=jax.ShapeDtypeStruct((2, NP, width), jnp.float32),
        scratch_types=[
            pltpu.VMEM_SHARED((NP, width), jnp.float32),
            pltpu.VMEM((kmax, BLK), jnp.int32),
            pltpu.VMEM((kmax, BLK), jnp.int32),
            pltpu.VMEM((BLK, width), jnp.float32),
            pltpu.VMEM((BLK, width), jnp.float32),
            pltpu.SemaphoreType.DMA,
            pltpu.SemaphoreType.DMA,
        ],
    )
    def spmm(y_hbm, src_hbm, dst_hbm, zeros_hbm, out_hbm,
             acc_sh, src_v, dst_v, rows0_v, rows1_v, sem0, sem1):
        c = lax.axis_index("c")
        s = lax.axis_index("s")

        def stage(k, base):
            pltpu.sync_copy(zeros_hbm.at[pl.ds(s * rows_per, rows_per)],
                            acc_sh.at[pl.ds(s * rows_per, rows_per)])
            pltpu.sync_copy(src_hbm.at[pl.ds(base + s * k, k)], src_v.at[pl.ds(0, k)])
            pltpu.sync_copy(dst_hbm.at[pl.ds(base + s * k, k)], dst_v.at[pl.ds(0, k)])

        @pl.when(c == 0)
        def _():
            stage(k0, 0)

        @pl.when(c == 1)
        def _():
            stage(k1, NS * k0)

        plsc.subcore_barrier()

        def run(k):
            def start(j, buf, sem):
                pltpu.async_copy(y_hbm.at[dst_v.at[j]], buf, sem)

            def wait(buf, sem):
                pltpu.make_async_copy(y_hbm.at[dst_v.at[0]], buf, sem).wait()

            def scat(buf, j):
                pltpu.sync_copy(buf, acc_sh.at[src_v.at[j]], add=True)

            start(0, rows0_v, sem0)

            def body(jj, carry):
                j0 = 2 * jj
                start(j0 + 1, rows1_v, sem1)
                wait(rows0_v, sem0)
                scat(rows0_v, j0)
                start(j0 + 2, rows0_v, sem0)
                wait(rows1_v, sem1)
                scat(rows1_v, j0 + 1)
                return carry

            lax.fori_loop(0, (k - 1) // 2, body, 0)
            wait(rows0_v, sem0)
            scat(rows0_v, k - 1)

        @pl.when(c == 0)
        def _():
            run(k0)

        @pl.when(c == 1)
        def _():
            run(k1)

        plsc.subcore_barrier()
        pltpu.sync_copy(acc_sh.at[pl.ds(s * rows_per, rows_per)],
                        out_hbm.at[c, pl.ds(s * rows_per, rows_per)])

    return spmm


def kernel(x, edge_index, W0, b0, W1, b1, W2, b2, Wf, bf):
    _spmm96 = _make_spmm(96, K0_96, K1_96)
    _spmm48 = _make_spmm(48, K0_48, K1_48)
    f32 = jnp.float32
    src, dst = edge_index[0], edge_index[1]
    pad_src = (N + jnp.arange(E_PAD - E, dtype=jnp.int32) % (NP - N)).astype(jnp.int32)
    pad_dst = jnp.full((E_PAD - E,), N, jnp.int32)
    srcf = jnp.concatenate([src, pad_src]).reshape(NS * NBT, BLK)
    dstf = jnp.concatenate([dst, pad_dst]).reshape(NS * NBT, BLK)
    dstf2 = dstf * 2

    x_p = jnp.zeros((NP, D), f32).at[:N].set(x)
    W_aug = jnp.zeros((D, 96), f32).at[:, 0:44].set(W2).at[:, 44:86].set(W1)
    b_aug = jnp.zeros((1, 96), f32).at[0, 95].set(1.0)
    W0p = jnp.zeros((D, 48), f32).at[:, 0:42].set(W0)
    b0p = jnp.zeros((1, 48), f32).at[0, 0:42].set(b0)
    b1p = jnp.zeros((1, 96), f32).at[0, 44:86].set(b1)
    b2p = jnp.zeros((1, 48), f32).at[0, 0:44].set(b2)
    Wf0p = jnp.zeros((48, 8), f32).at[0:42, 0:2].set(Wf[0:42])
    Wf1p = jnp.zeros((96, 8), f32).at[44:86, 0:2].set(Wf[42:84])
    Wf2p = jnp.zeros((48, 8), f32).at[0:44, 0:2].set(Wf[84:128])
    bfp = jnp.zeros((1, 8), f32).at[0, 0:2].set(bf)
    zeros96 = jnp.zeros((NP, 96), f32)
    zeros48 = jnp.zeros((NP, 48), f32)

    y_aug = _k1a(x_p, W_aug, b_aug)
    zp = _spmm96(y_aug, srcf, dstf, zeros96)
    h0r = _k1b(x_p, W0p, b0p)
    z = _k3a(zp)
    tp = _spmm48(z.reshape(2 * NP, 48), srcf, dstf2, zeros48)
    h1r, d2b = _k3b(z, b1p)
    s01 = _k5a(h0r, h1r, Wf0p, Wf1p, bfp)
    out = _k5b(s01, tp, d2b, b2p, Wf2p)
    return out[:N, 0:2]


# speedup vs baseline: 1.0781x; 1.0781x over previous
import functools

import jax
import jax.numpy as jnp
from jax import lax
from jax.experimental import pallas as pl
from jax.experimental.pallas import tpu as pltpu
from jax.experimental.pallas import tpu_sc as plsc

N = 10000
D = 128
NP = 10240
E = 320000
NS = 16
BLK = 128
NBT = 158
E_PAD = NS * NBT * BLK
BM = 512
GRID = NP // BM
K0_96, K0_48 = 97, 99
K1_96, K1_48 = NBT - K0_96, NBT - K0_48


def _k1a_body(x_ref, wa_ref, ba_ref, y_ref):
    y_ref[...] = jnp.dot(x_ref[...], wa_ref[...],
                         preferred_element_type=jnp.float32) + ba_ref[...]


def _k1a(x_p, W_aug, b_aug):
    return pl.pallas_call(
        _k1a_body,
        grid=(GRID,),
        in_specs=[
            pl.BlockSpec((BM, D), lambda i: (i, 0)),
            pl.BlockSpec((D, 96), lambda i: (0, 0)),
            pl.BlockSpec((1, 96), lambda i: (0, 0)),
        ],
        out_specs=pl.BlockSpec((BM, 96), lambda i: (i, 0)),
        out_shape=jax.ShapeDtypeStruct((NP, 96), jnp.float32),
    )(x_p, W_aug, b_aug)


def _k1b_body(x_ref, w0_ref, b0_ref, h0_ref):
    h0_ref[...] = jnp.maximum(
        jnp.dot(x_ref[...], w0_ref[...], preferred_element_type=jnp.float32)
        + b0_ref[...], 0.0)


def _k1b(x_p, W0p, b0p):
    return pl.pallas_call(
        _k1b_body,
        grid=(GRID,),
        in_specs=[
            pl.BlockSpec((BM, D), lambda i: (i, 0)),
            pl.BlockSpec((D, 48), lambda i: (0, 0)),
            pl.BlockSpec((1, 48), lambda i: (0, 0)),
        ],
        out_specs=pl.BlockSpec((BM, 48), lambda i: (i, 0)),
        out_shape=jax.ShapeDtypeStruct((NP, 48), jnp.float32),
    )(x_p, W0p, b0p)


def _make_sc_combine():
    stripe = NP // 32
    mesh = plsc.VectorSubcoreMesh(core_axis_name="c", subcore_axis_name="s")

    @functools.partial(
        pl.kernel,
        mesh=mesh,
        compiler_params=pltpu.CompilerParams(use_tc_tiling_on_sc=False),
        out_type=jax.ShapeDtypeStruct((NP, 96), jnp.float32),
        scratch_types=[
            pltpu.VMEM((stripe, 96), jnp.float32),
            pltpu.VMEM((stripe, 96), jnp.float32),
        ],
    )
    def comb(zp_hbm, out_hbm, a_v, b_v):
        c = lax.axis_index("c")
        s = lax.axis_index("s")
        base = (c * NS + s) * stripe
        pltpu.sync_copy(zp_hbm.at[0, pl.ds(base, stripe)], a_v)
        pltpu.sync_copy(zp_hbm.at[1, pl.ds(base, stripe)], b_v)

        def body(r, carry):
            for cc in range(6):
                sl = pl.ds(cc * 16, 16)
                a_v[r, sl] = a_v[r, sl] + b_v[r, sl]
            return carry

        lax.fori_loop(0, stripe, body, 0)
        pltpu.sync_copy(a_v, out_hbm.at[pl.ds(base, stripe)])

    return comb


def _k3b_body(z_ref, b1p_ref, h1r_ref, d2b_ref):
    z = z_ref[...]
    dinv = 1.0 / jnp.maximum(z[:, 95:96], 1.0)
    h1r_ref[...] = jnp.maximum(z * dinv + b1p_ref[...], 0.0)
    d2b_ref[...] = jnp.broadcast_to(dinv * dinv, (BM, 48))


def _k3b(z, b1p):
    return pl.pallas_call(
        _k3b_body,
        grid=(GRID,),
        in_specs=[
            pl.BlockSpec((BM, 96), lambda i: (i, 0)),
            pl.BlockSpec((1, 96), lambda i: (0, 0)),
        ],
        out_specs=[
            pl.BlockSpec((BM, 96), lambda i: (i, 0)),
            pl.BlockSpec((BM, 48), lambda i: (i, 0)),
        ],
        out_shape=[
            jax.ShapeDtypeStruct((NP, 96), jnp.float32),
            jax.ShapeDtypeStruct((NP, 48), jnp.float32),
        ],
    )(z, b1p)


def _k5a_body(h0r_ref, h1r_ref, wf0_ref, wf1_ref, bf_ref, s_ref):
    s_ref[...] = (
        jnp.dot(h0r_ref[...], wf0_ref[...], preferred_element_type=jnp.float32)
        + jnp.dot(h1r_ref[...], wf1_ref[...], preferred_element_type=jnp.float32)
        + bf_ref[...])


def _k5a(h0r, h1r, Wf0p, Wf1p, bfp):
    return pl.pallas_call(
        _k5a_body,
        grid=(GRID,),
        in_specs=[
            pl.BlockSpec((BM, 48), lambda i: (i, 0)),
            pl.BlockSpec((BM, 96), lambda i: (i, 0)),
            pl.BlockSpec((48, 8), lambda i: (0, 0)),
            pl.BlockSpec((96, 8), lambda i: (0, 0)),
            pl.BlockSpec((1, 8), lambda i: (0, 0)),
        ],
        out_specs=pl.BlockSpec((BM, 8), lambda i: (i, 0)),
        out_shape=jax.ShapeDtypeStruct((NP, 8), jnp.float32),
    )(h0r, h1r, Wf0p, Wf1p, bfp)


def _k5b_body(s_ref, tp_ref, d2b_ref, b2p_ref, wf2_ref, out_ref):
    t = tp_ref[0] + tp_ref[1]
    h2r = jnp.maximum(t * d2b_ref[...] + b2p_ref[...], 0.0)
    out_ref[...] = s_ref[...] + jnp.dot(
        h2r, wf2_ref[...], preferred_element_type=jnp.float32)


def _k5b(s01, tp, d2b, b2p, Wf2p):
    return pl.pallas_call(
        _k5b_body,
        grid=(GRID,),
        in_specs=[
            pl.BlockSpec((BM, 8), lambda i: (i, 0)),
            pl.BlockSpec((2, BM, 48), lambda i: (0, i, 0)),
            pl.BlockSpec((BM, 48), lambda i: (i, 0)),
            pl.BlockSpec((1, 48), lambda i: (0, 0)),
            pl.BlockSpec((48, 8), lambda i: (0, 0)),
        ],
        out_specs=pl.BlockSpec((BM, 8), lambda i: (i, 0)),
        out_shape=jax.ShapeDtypeStruct((NP, 8), jnp.float32),
    )(s01, tp, d2b, b2p, Wf2p)


def _make_spmm(width, k0, k1):
    rows_per = NP // NS
    kmax = max(k0, k1)
    mesh = plsc.VectorSubcoreMesh(core_axis_name="c", subcore_axis_name="s")

    @functools.partial(
        pl.kernel,
        mesh=mesh,
        compiler_params=pltpu.CompilerParams(use_tc_tiling_on_sc=False),
        out_type=jax.ShapeDtypeStruct((2, NP, width), jnp.float32),
        scratch_types=[
            pltpu.VMEM_SHARED((NP, width), jnp.float32),
            pltpu.VMEM((kmax, BLK), jnp.int32),
            pltpu.VMEM((kmax, BLK), jnp.int32),
            pltpu.VMEM((BLK, width), jnp.float32),
            pltpu.VMEM((BLK, width), jnp.float32),
            pltpu.SemaphoreType.DMA,
            pltpu.SemaphoreType.DMA,
        ],
    )
    def spmm(y_hbm, src_hbm, dst_hbm, zeros_hbm, out_hbm,
             acc_sh, src_v, dst_v, rows0_v, rows1_v, sem0, sem1):
        c = lax.axis_index("c")
        s = lax.axis_index("s")

        def stage(k, base):
            pltpu.sync_copy(zeros_hbm.at[pl.ds(s * rows_per, rows_per)],
                            acc_sh.at[pl.ds(s * rows_per, rows_per)])
            pltpu.sync_copy(src_hbm.at[pl.ds(base + s * k, k)], src_v.at[pl.ds(0, k)])
            pltpu.sync_copy(dst_hbm.at[pl.ds(base + s * k, k)], dst_v.at[pl.ds(0, k)])

        @pl.when(c == 0)
        def _():
            stage(k0, 0)

        @pl.when(c == 1)
        def _():
            stage(k1, NS * k0)

        plsc.subcore_barrier()

        def run(k):
            def start(j, buf, sem):
                pltpu.async_copy(y_hbm.at[dst_v.at[j]], buf, sem)

            def wait(buf, sem):
                pltpu.make_async_copy(y_hbm.at[dst_v.at[0]], buf, sem).wait()

            def scat(buf, j):
                pltpu.sync_copy(buf, acc_sh.at[src_v.at[j]], add=True)

            start(0, rows0_v, sem0)

            def body(jj, carry):
                j0 = 2 * jj
                start(j0 + 1, rows1_v, sem1)
                wait(rows0_v, sem0)
                scat(rows0_v, j0)
                start(j0 + 2, rows0_v, sem0)
                wait(rows1_v, sem1)
                scat(rows1_v, j0 + 1)
                return carry

            lax.fori_loop(0, (k - 1) // 2, body, 0)
            wait(rows0_v, sem0)
            scat(rows0_v, k - 1)

        @pl.when(c == 0)
        def _():
            run(k0)

        @pl.when(c == 1)
        def _():
            run(k1)

        plsc.subcore_barrier()
        pltpu.sync_copy(acc_sh.at[pl.ds(s * rows_per, rows_per)],
                        out_hbm.at[c, pl.ds(s * rows_per, rows_per)])

    return spmm


def kernel(x, edge_index, W0, b0, W1, b1, W2, b2, Wf, bf):
    _spmm96 = _make_spmm(96, K0_96, K1_96)
    _spmm48 = _make_spmm(48, K0_48, K1_48)
    f32 = jnp.float32
    src, dst = edge_index[0], edge_index[1]
    pad_src = (N + jnp.arange(E_PAD - E, dtype=jnp.int32) % (NP - N)).astype(jnp.int32)
    pad_dst = jnp.full((E_PAD - E,), N, jnp.int32)
    srcf = jnp.concatenate([src, pad_src]).reshape(NS * NBT, BLK)
    dstf = jnp.concatenate([dst, pad_dst]).reshape(NS * NBT, BLK)
    dstf2 = dstf * 2

    x_p = jnp.zeros((NP, D), f32).at[:N].set(x)
    W_aug = jnp.zeros((D, 96), f32).at[:, 0:44].set(W2).at[:, 44:86].set(W1)
    b_aug = jnp.zeros((1, 96), f32).at[0, 95].set(1.0)
    W0p = jnp.zeros((D, 48), f32).at[:, 0:42].set(W0)
    b0p = jnp.zeros((1, 48), f32).at[0, 0:42].set(b0)
    b1p = jnp.zeros((1, 96), f32).at[0, 44:86].set(b1)
    b2p = jnp.zeros((1, 48), f32).at[0, 0:44].set(b2)
    Wf0p = jnp.zeros((48, 8), f32).at[0:42, 0:2].set(Wf[0:42])
    Wf1p = jnp.zeros((96, 8), f32).at[44:86, 0:2].set(Wf[42:84])
    Wf2p = jnp.zeros((48, 8), f32).at[0:44, 0:2].set(Wf[84:128])
    bfp = jnp.zeros((1, 8), f32).at[0, 0:2].set(bf)
    zeros96 = jnp.zeros((NP, 96), f32)
    zeros48 = jnp.zeros((NP, 48), f32)

    y_aug = _k1a(x_p, W_aug, b_aug)
    zp = _spmm96(y_aug, srcf, dstf, zeros96)
    h0r = _k1b(x_p, W0p, b0p)
    z = _make_sc_combine()(zp)
    tp = _spmm48(z.reshape(2 * NP, 48), srcf, dstf2, zeros48)
    h1r, d2b = _k3b(z, b1p)
    s01 = _k5a(h0r, h1r, Wf0p, Wf1p, bfp)
    out = _k5b(s01, tp, d2b, b2p, Wf2p)
    return out[:N, 0:2]
